# baseline (speedup 1.0000x reference)
import functools

import jax
import jax.numpy as jnp
from jax import lax
from jax.experimental import pallas as pl
from jax.experimental.pallas import tpu as pltpu
from jax.experimental.pallas import tpu_sc as plsc

N = 320000
D = 128
S = 1024
HALF = S // 2

NC = 2
NS = 16

CHUNK = 128
NCHUNKS = N // CHUNK
BLK_CHUNKS = 2
BLK = BLK_CHUNKS * CHUNK
LANES = 16
NSEG_V = D // LANES
TRASH = S
INIT_ROWS = S // NS
PUB_ROWS = HALF // NS

IB_W = 192
IB_TAIL = 4
IB_LAST = 2304

_mesh = plsc.VectorSubcoreMesh(core_axis_name="c", subcore_axis_name="s")


@functools.partial(
    pl.kernel,
    mesh=_mesh,
    out_type=jax.ShapeDtypeStruct((S, D), jnp.float32),
    scratch_types=[
        pltpu.VMEM((2, BLK, D), jnp.float32),
        pltpu.VMEM((IB_W + IB_TAIL, CHUNK), jnp.int32),
        pltpu.VMEM((INIT_ROWS, D), jnp.float32),
        pltpu.VMEM((LANES, D), jnp.float32),
        pltpu.VMEM((LANES,), jnp.int32),
        pltpu.VMEM((LANES,), jnp.int32),
        pltpu.VMEM_SHARED((S + 8, D), jnp.float32),
        pltpu.SemaphoreType.DMA,
        pltpu.SemaphoreType.DMA,
    ],
)
def _segsum_sc(v_hbm, batch_hbm, u_hbm, zeros_hbm, p_hbm, out_hbm,
               vbuf, ibuf, obuf, sbuf, idbuf, pbuf, acc, sem0, sem1):
    c = lax.axis_index("c")
    s = lax.axis_index("s")
    lane_iota = lax.iota(jnp.int32, LANES)

    pltpu.sync_copy(p_hbm, pbuf)
    p = pbuf[...][0]
    n0 = (p + CHUNK - 1) // CHUNK
    s1f = p // CHUNK
    my_n = jnp.where(c == 0, n0, NCHUNKS - s1f)
    my_base = jnp.where(c == 0, 0, s1f)
    q = my_n // NS
    r = my_n - q * NS
    cnt = q + (s < r).astype(jnp.int32)
    tstart = my_base + s * q + jnp.minimum(s, r)
    row0 = tstart * CHUNK

    def vload(g, slot, sem):
        off = jnp.minimum(row0 + g * BLK, N - BLK)
        return pltpu.make_async_copy(
            v_hbm.at[pl.ds(off, BLK), :], vbuf.at[slot], sem)

    vload(0, 0, sem0).start()
    vload(1, 1, sem1).start()
    ib_start = pl.multiple_of(
        jnp.minimum((tstart // 8) * 8, IB_LAST), 8)
    ib_off = tstart - ib_start
    pltpu.sync_copy(batch_hbm.at[pl.ds(ib_start, IB_W)],
                    ibuf.at[pl.ds(0, IB_W)])
    pltpu.sync_copy(batch_hbm.at[pl.ds(NCHUNKS - IB_TAIL, IB_TAIL)],
                    ibuf.at[pl.ds(IB_W, IB_TAIL)])

    init_u = (s < NS // 2) == (c == 0)

    @pl.when(init_u)
    def _():
        pltpu.sync_copy(u_hbm.at[pl.ds(s * INIT_ROWS, INIT_ROWS), :], obuf)

    @pl.when(jnp.logical_not(init_u))
    def _():
        pltpu.sync_copy(zeros_hbm, obuf)

    pltpu.sync_copy(obuf, acc.at[pl.ds(s * INIT_ROWS, INIT_ROWS)])
    plsc.subcore_barrier()

    def reduce_chunk(slot, base):
        def rbody(rr, accs):
            out = accs
            for ru in range(4):
                row = base + 4 * rr + ru
                out = tuple(
                    a + jnp.reshape(
                        vbuf[slot, pl.ds(row, 1), pl.ds(j * LANES, LANES)],
                        (LANES,))
                    for j, a in enumerate(out))
            return out

        zero = tuple(jnp.zeros((LANES,), jnp.float32) for _ in range(NSEG_V))
        return lax.fori_loop(0, CHUNK // 4, rbody, zero)

    def fold_chunk(k, irow, slot, base):
        fv = jnp.reshape(ibuf[pl.ds(irow, 1), pl.ds(0, LANES)], (LANES,))
        lv = jnp.reshape(
            ibuf[pl.ds(irow, 1), pl.ds(CHUNK - LANES, LANES)], (LANES,))
        single = fv[0] == lv[LANES - 1]

        @pl.when(single)
        def _():
            sums = reduce_chunk(slot, base)
            for j in range(NSEG_V):
                sbuf[k, pl.ds(j * LANES, LANES)] = sums[j]
            idbuf[...] = jnp.where(lane_iota == k, fv, idbuf[...])

        @pl.when(jnp.logical_not(single))
        def _():
            pltpu.sync_copy(vbuf.at[slot, pl.ds(base, CHUNK)],
                            acc.at[ibuf.at[irow]], add=True)

    def fold_block(g, slot):
        idbuf[...] = jnp.full((LANES,), TRASH, jnp.int32)
        for k in range(BLK_CHUNKS):
            fold_chunk(k, ib_off + g * BLK_CHUNKS + k, slot, k * CHUNK)
        pltpu.sync_copy(sbuf, acc.at[idbuf], add=True)

    nblk = cnt // BLK_CHUNKS

    def body(gg, carry):
        g0 = 2 * gg
        vload(g0, 0, sem0).wait()
        fold_block(g0, 0)

        @pl.when(g0 + 2 < nblk)
        def _():
            vload(g0 + 2, 0, sem0).start()

        vload(g0 + 1, 1, sem1).wait()
        fold_block(g0 + 1, 1)

        @pl.when(g0 + 3 < nblk)
        def _():
            vload(g0 + 3, 1, sem1).start()

        return carry

    lax.fori_loop(0, nblk // 2, body, 0)

    @pl.when(nblk % 2 == 1)
    def _():
        vload(nblk - 1, 0, sem0).wait()
        fold_block(nblk - 1, 0)

    @pl.when(nblk == 0)
    def _():
        vload(0, 0, sem0).wait()

    @pl.when(nblk <= 1)
    def _():
        vload(1, 1, sem1).wait()

    @pl.when(cnt % 2 == 1)
    def _():
        last = cnt - 1
        pltpu.sync_copy(
            v_hbm.at[pl.ds((tstart + last) * CHUNK, CHUNK), :],
            vbuf.at[0, pl.ds(0, CHUNK)])
        idbuf[...] = jnp.full((LANES,), TRASH, jnp.int32)
        fold_chunk(0, ib_off + last, 0, 0)
        pltpu.sync_copy(sbuf, acc.at[idbuf], add=True)

    plsc.subcore_barrier()

    orow = c * HALF + s * PUB_ROWS
    pltpu.sync_copy(acc.at[pl.ds(orow, PUB_ROWS)],
                    obuf.at[pl.ds(0, PUB_ROWS)])
    pltpu.sync_copy(obuf.at[pl.ds(0, PUB_ROWS)],
                    out_hbm.at[pl.ds(orow, PUB_ROWS), :])


def kernel(u, v, batch):
    batch32 = batch.astype(jnp.int32)
    batch2d = batch32.reshape(NCHUNKS, CHUNK)
    p = jnp.searchsorted(batch32, HALF).astype(jnp.int32)
    p_arr = jnp.full((LANES,), p, jnp.int32)
    zeros = jnp.zeros((INIT_ROWS, D), jnp.float32)
    return _segsum_sc(v, batch2d, u, zeros, p_arr)


# speedup vs baseline: 1.2932x; 1.2932x over previous
import functools

import jax
import jax.numpy as jnp
from jax import lax
from jax.experimental import pallas as pl
from jax.experimental.pallas import tpu as pltpu
from jax.experimental.pallas import tpu_sc as plsc

N = 320000
D = 128
S = 1024
HALF = S // 2

NC = 2
NS = 16

CHUNK = 128
NCHUNKS = N // CHUNK
BLK_CHUNKS = 2
BLK = BLK_CHUNKS * CHUNK
LANES = 16
NSEG_V = D // LANES
TRASH = S
INIT_ROWS = S // NS
PUB_ROWS = HALF // NS

IB_W = 192
IB_TAIL = 4
IB_LAST = 2304

_mesh = plsc.VectorSubcoreMesh(core_axis_name="c", subcore_axis_name="s")


@functools.partial(
    pl.kernel,
    mesh=_mesh,
    out_type=jax.ShapeDtypeStruct((S, D), jnp.float32),
    scratch_types=[
        pltpu.VMEM((2, BLK, D), jnp.float32),
        pltpu.VMEM((IB_W + IB_TAIL, CHUNK), jnp.int32),
        pltpu.VMEM((INIT_ROWS, D), jnp.float32),
        pltpu.VMEM((LANES, D), jnp.float32),
        pltpu.VMEM((LANES,), jnp.int32),
        pltpu.VMEM((LANES,), jnp.int32),
        pltpu.VMEM_SHARED((S + 8, D), jnp.float32),
        pltpu.SemaphoreType.DMA,
        pltpu.SemaphoreType.DMA,
    ],
)
def _segsum_sc(v_hbm, batch_hbm, u_hbm, zeros_hbm, p_hbm, out_hbm,
               vbuf, ibuf, obuf, sbuf, idbuf, pbuf, acc, sem0, sem1):
    c = lax.axis_index("c")
    s = lax.axis_index("s")
    lane_iota = lax.iota(jnp.int32, LANES)

    pltpu.sync_copy(p_hbm, pbuf)
    pv = pbuf[...]
    n0 = pv[0]
    s1f = pv[1]
    my_n = jnp.where(c == 0, n0, NCHUNKS - s1f)
    my_base = jnp.where(c == 0, 0, s1f)
    q = my_n // NS
    r = my_n - q * NS
    cnt = q + (s < r).astype(jnp.int32)
    tstart = my_base + s * q + jnp.minimum(s, r)
    row0 = tstart * CHUNK

    def vload(g, slot, sem):
        off = jnp.minimum(row0 + g * BLK, N - BLK)
        return pltpu.make_async_copy(
            v_hbm.at[pl.ds(off, BLK), :], vbuf.at[slot], sem)

    vload(0, 0, sem0).start()
    vload(1, 1, sem1).start()
    ib_start = pl.multiple_of(
        jnp.minimum((tstart // 8) * 8, IB_LAST), 8)
    ib_off = tstart - ib_start
    pltpu.sync_copy(batch_hbm.at[pl.ds(ib_start, IB_W)],
                    ibuf.at[pl.ds(0, IB_W)])
    pltpu.sync_copy(batch_hbm.at[pl.ds(NCHUNKS - IB_TAIL, IB_TAIL)],
                    ibuf.at[pl.ds(IB_W, IB_TAIL)])

    init_u = (s < NS // 2) == (c == 0)

    @pl.when(init_u)
    def _():
        pltpu.sync_copy(u_hbm.at[pl.ds(s * INIT_ROWS, INIT_ROWS), :], obuf)

    @pl.when(jnp.logical_not(init_u))
    def _():
        pltpu.sync_copy(zeros_hbm, obuf)

    pltpu.sync_copy(obuf, acc.at[pl.ds(s * INIT_ROWS, INIT_ROWS)])
    plsc.subcore_barrier()

    def reduce_chunk(slot, base):
        def rbody(rr, accs):
            out = accs
            for ru in range(4):
                row = base + 4 * rr + ru
                out = tuple(
                    a + jnp.reshape(
                        vbuf[slot, pl.ds(row, 1), pl.ds(j * LANES, LANES)],
                        (LANES,))
                    for j, a in enumerate(out))
            return out

        zero = tuple(jnp.zeros((LANES,), jnp.float32) for _ in range(NSEG_V))
        return lax.fori_loop(0, CHUNK // 4, rbody, zero)

    def fold_chunk(k, irow, slot, base):
        fv = jnp.reshape(ibuf[pl.ds(irow, 1), pl.ds(0, LANES)], (LANES,))
        lv = jnp.reshape(
            ibuf[pl.ds(irow, 1), pl.ds(CHUNK - LANES, LANES)], (LANES,))
        single = fv[0] == lv[LANES - 1]

        @pl.when(single)
        def _():
            sums = reduce_chunk(slot, base)
            for j in range(NSEG_V):
                sbuf[k, pl.ds(j * LANES, LANES)] = sums[j]
            idbuf[...] = jnp.where(lane_iota == k, fv, idbuf[...])

        @pl.when(jnp.logical_not(single))
        def _():
            pltpu.sync_copy(vbuf.at[slot, pl.ds(base, CHUNK)],
                            acc.at[ibuf.at[irow]], add=True)

    def fold_block(g, slot):
        idbuf[...] = jnp.full((LANES,), TRASH, jnp.int32)
        for k in range(BLK_CHUNKS):
            fold_chunk(k, ib_off + g * BLK_CHUNKS + k, slot, k * CHUNK)
        pltpu.sync_copy(sbuf, acc.at[idbuf], add=True)

    nblk = cnt // BLK_CHUNKS

    def body(gg, carry):
        g0 = 2 * gg
        vload(g0, 0, sem0).wait()
        fold_block(g0, 0)

        @pl.when(g0 + 2 < nblk)
        def _():
            vload(g0 + 2, 0, sem0).start()

        vload(g0 + 1, 1, sem1).wait()
        fold_block(g0 + 1, 1)

        @pl.when(g0 + 3 < nblk)
        def _():
            vload(g0 + 3, 1, sem1).start()

        return carry

    lax.fori_loop(0, nblk // 2, body, 0)

    @pl.when(nblk % 2 == 1)
    def _():
        vload(nblk - 1, 0, sem0).wait()
        fold_block(nblk - 1, 0)

    @pl.when(nblk == 0)
    def _():
        vload(0, 0, sem0).wait()

    @pl.when(nblk <= 1)
    def _():
        vload(1, 1, sem1).wait()

    @pl.when(cnt % 2 == 1)
    def _():
        last = cnt - 1
        pltpu.sync_copy(
            v_hbm.at[pl.ds((tstart + last) * CHUNK, CHUNK), :],
            vbuf.at[0, pl.ds(0, CHUNK)])
        idbuf[...] = jnp.full((LANES,), TRASH, jnp.int32)
        fold_chunk(0, ib_off + last, 0, 0)
        pltpu.sync_copy(sbuf, acc.at[idbuf], add=True)

    plsc.subcore_barrier()

    orow = c * HALF + s * PUB_ROWS
    pltpu.sync_copy(acc.at[pl.ds(orow, PUB_ROWS)],
                    obuf.at[pl.ds(0, PUB_ROWS)])
    pltpu.sync_copy(obuf.at[pl.ds(0, PUB_ROWS)],
                    out_hbm.at[pl.ds(orow, PUB_ROWS), :])


def kernel(u, v, batch):
    batch32 = batch.astype(jnp.int32)
    batch2d = batch32.reshape(NCHUNKS, CHUNK)
    n0 = jnp.sum((batch2d[:, 0] < HALF).astype(jnp.int32))
    s1f = NCHUNKS - jnp.sum((batch2d[:, CHUNK - 1] >= HALF).astype(jnp.int32))
    p_arr = jnp.stack([n0, s1f])[
        jnp.minimum(jnp.arange(LANES), 1)].astype(jnp.int32)
    zeros = jnp.zeros((INIT_ROWS, D), jnp.float32)
    return _segsum_sc(v, batch2d, u, zeros, p_arr)


# speedup vs baseline: 1.3182x; 1.0194x over previous
import functools

import jax
import jax.numpy as jnp
from jax import lax
from jax.experimental import pallas as pl
from jax.experimental.pallas import tpu as pltpu
from jax.experimental.pallas import tpu_sc as plsc

N = 320000
D = 128
S = 1024
HALF = S // 2

NC = 2
NS = 16

CHUNK = 128
NCHUNKS = N // CHUNK
BLK_CHUNKS = 2
BLK = BLK_CHUNKS * CHUNK
LANES = 16
NSEG_V = D // LANES
TRASH = S
INIT_ROWS = S // NS
PUB_ROWS = HALF // NS

IB_W = 192
IB_TAIL = 4
IB_LAST = 2304

_mesh = plsc.VectorSubcoreMesh(core_axis_name="c", subcore_axis_name="s")


@functools.partial(
    pl.kernel,
    mesh=_mesh,
    out_type=jax.ShapeDtypeStruct((S, D), jnp.float32),
    scratch_types=[
        pltpu.VMEM((2, BLK, D), jnp.float32),
        pltpu.VMEM((IB_W + IB_TAIL, CHUNK), jnp.int32),
        pltpu.VMEM((INIT_ROWS, D), jnp.float32),
        pltpu.VMEM((LANES, D), jnp.float32),
        pltpu.VMEM((LANES,), jnp.int32),
        pltpu.VMEM((LANES,), jnp.int32),
        pltpu.VMEM_SHARED((S + 8, D), jnp.float32),
        pltpu.SemaphoreType.DMA,
        pltpu.SemaphoreType.DMA,
    ],
)
def _segsum_sc(v_hbm, batch_hbm, u_hbm, zeros_hbm, p_hbm, out_hbm,
               vbuf, ibuf, obuf, sbuf, idbuf, pbuf, acc, sem0, sem1):
    c = lax.axis_index("c")
    s = lax.axis_index("s")
    lane_iota = lax.iota(jnp.int32, LANES)

    pltpu.sync_copy(p_hbm, pbuf)
    pv = pbuf[...]
    n0 = pv[0]
    s1f = pv[1]
    my_n = jnp.where(c == 0, n0, NCHUNKS - s1f)
    my_base = jnp.where(c == 0, 0, s1f)
    q = my_n // NS
    r = my_n - q * NS
    cnt = q + (s < r).astype(jnp.int32)
    tstart = my_base + s * q + jnp.minimum(s, r)
    row0 = tstart * CHUNK

    def vload(g, slot, sem):
        off = jnp.minimum(row0 + g * BLK, N - BLK)
        return pltpu.make_async_copy(
            v_hbm.at[pl.ds(off, BLK), :], vbuf.at[slot], sem)

    vload(0, 0, sem0).start()
    vload(1, 1, sem1).start()
    ib_start = pl.multiple_of(
        jnp.minimum((tstart // 8) * 8, IB_LAST), 8)
    ib_off = tstart - ib_start
    pltpu.sync_copy(batch_hbm.at[pl.ds(ib_start, IB_W)],
                    ibuf.at[pl.ds(0, IB_W)])
    pltpu.sync_copy(batch_hbm.at[pl.ds(NCHUNKS - IB_TAIL, IB_TAIL)],
                    ibuf.at[pl.ds(IB_W, IB_TAIL)])

    init_u = (s < NS // 2) == (c == 0)

    @pl.when(init_u)
    def _():
        pltpu.sync_copy(u_hbm.at[pl.ds(s * INIT_ROWS, INIT_ROWS), :], obuf)

    @pl.when(jnp.logical_not(init_u))
    def _():
        pltpu.sync_copy(zeros_hbm, obuf)

    pltpu.sync_copy(obuf, acc.at[pl.ds(s * INIT_ROWS, INIT_ROWS)])
    plsc.subcore_barrier()

    def reduce_chunk(slot, base):
        def rbody(rr, accs):
            out = accs
            for ru in range(4):
                row = base + 4 * rr + ru
                out = tuple(
                    a + jnp.reshape(
                        vbuf[slot, pl.ds(row, 1), pl.ds(j * LANES, LANES)],
                        (LANES,))
                    for j, a in enumerate(out))
            return out

        zero = tuple(jnp.zeros((LANES,), jnp.float32) for _ in range(NSEG_V))
        return lax.fori_loop(0, CHUNK // 4, rbody, zero)

    def fold_chunk(k, irow, slot, base):
        fv = jnp.reshape(ibuf[pl.ds(irow, 1), pl.ds(0, LANES)], (LANES,))
        lv = jnp.reshape(
            ibuf[pl.ds(irow, 1), pl.ds(CHUNK - LANES, LANES)], (LANES,))
        single = fv[0] == lv[LANES - 1]

        @pl.when(single)
        def _():
            sums = reduce_chunk(slot, base)
            for j in range(NSEG_V):
                sbuf[k, pl.ds(j * LANES, LANES)] = sums[j]
            idbuf[...] = jnp.where(lane_iota == k, fv, idbuf[...])

        @pl.when(jnp.logical_not(single))
        def _():
            pltpu.sync_copy(vbuf.at[slot, pl.ds(base, CHUNK)],
                            acc.at[ibuf.at[irow]], add=True)

    def fold_block(g, slot, kbase):
        for k in range(BLK_CHUNKS):
            fold_chunk(kbase + k, ib_off + g * BLK_CHUNKS + k, slot,
                       k * CHUNK)

    nblk = cnt // BLK_CHUNKS

    def body(gg, carry):
        g0 = 2 * gg
        idbuf[...] = jnp.full((LANES,), TRASH, jnp.int32)
        vload(g0, 0, sem0).wait()
        fold_block(g0, 0, 0)

        @pl.when(g0 + 2 < nblk)
        def _():
            vload(g0 + 2, 0, sem0).start()

        vload(g0 + 1, 1, sem1).wait()
        fold_block(g0 + 1, 1, BLK_CHUNKS)

        @pl.when(g0 + 3 < nblk)
        def _():
            vload(g0 + 3, 1, sem1).start()

        pltpu.sync_copy(sbuf, acc.at[idbuf], add=True)
        return carry

    lax.fori_loop(0, nblk // 2, body, 0)

    @pl.when(nblk % 2 == 1)
    def _():
        vload(nblk - 1, 0, sem0).wait()
        idbuf[...] = jnp.full((LANES,), TRASH, jnp.int32)
        fold_block(nblk - 1, 0, 0)
        pltpu.sync_copy(sbuf, acc.at[idbuf], add=True)

    @pl.when(nblk == 0)
    def _():
        vload(0, 0, sem0).wait()

    @pl.when(nblk <= 1)
    def _():
        vload(1, 1, sem1).wait()

    @pl.when(cnt % 2 == 1)
    def _():
        last = cnt - 1
        pltpu.sync_copy(
            v_hbm.at[pl.ds((tstart + last) * CHUNK, CHUNK), :],
            vbuf.at[0, pl.ds(0, CHUNK)])
        idbuf[...] = jnp.full((LANES,), TRASH, jnp.int32)
        fold_chunk(0, ib_off + last, 0, 0)
        pltpu.sync_copy(sbuf, acc.at[idbuf], add=True)

    plsc.subcore_barrier()

    orow = c * HALF + s * PUB_ROWS
    pltpu.sync_copy(acc.at[pl.ds(orow, PUB_ROWS)],
                    obuf.at[pl.ds(0, PUB_ROWS)])
    pltpu.sync_copy(obuf.at[pl.ds(0, PUB_ROWS)],
                    out_hbm.at[pl.ds(orow, PUB_ROWS), :])


def kernel(u, v, batch):
    batch32 = batch.astype(jnp.int32)
    batch2d = batch32.reshape(NCHUNKS, CHUNK)
    n0 = jnp.sum((batch2d[:, 0] < HALF).astype(jnp.int32))
    s1f = NCHUNKS - jnp.sum((batch2d[:, CHUNK - 1] >= HALF).astype(jnp.int32))
    p_arr = jnp.stack([n0, s1f])[
        jnp.minimum(jnp.arange(LANES), 1)].astype(jnp.int32)
    zeros = jnp.zeros((INIT_ROWS, D), jnp.float32)
    return _segsum_sc(v, batch2d, u, zeros, p_arr)
